# baseline (speedup 1.0000x reference)
import functools

import jax
import jax.numpy as jnp
from jax import lax
from jax.experimental import pallas as pl
from jax.experimental.pallas import tpu as pltpu
from jax.experimental.pallas import tpu_sc as plsc

_NQ = 128
_L = 16


def _sel_body(intra_ref, scores_ref, x_ref, newX_ref, idx_ref, gidx_ref):
    ns = intra_ref.shape[-1]
    n = scores_ref.shape[-1]
    kc = ns // 2
    m = kc + _NQ
    bb = pl.program_id(0)

    it_row = intra_ref[0]
    s_row = scores_ref[0]
    it_col = it_row.reshape(ns, 1)
    j_col = jax.lax.broadcasted_iota(jnp.int32, (ns, 1), 0)
    j_row = jax.lax.broadcasted_iota(jnp.int32, (1, ns), 1)

    CH = 384
    ones_row = jnp.full((1, ns), 1.0, dtype=jnp.bfloat16)
    rank_chunks = []
    for c0 in range(0, ns, CH):
        it_i = jax.lax.slice(it_row, (0, c0), (1, c0 + CH))
        i_row = jax.lax.broadcasted_iota(jnp.int32, (1, CH), 1) + c0
        less = it_col < it_i
        tie = (it_col == it_i) & (j_col < i_row)
        mask = (less | tie).astype(jnp.bfloat16)
        rank_chunks.append(jax.lax.dot_general(
            ones_row, mask, (((1,), (0,)), ((), ())),
            preferred_element_type=jnp.float32))
    rank_row = jnp.concatenate(rank_chunks, axis=1).astype(jnp.int32)

    s_supp_row = jax.lax.slice(s_row, (0, 0), (1, ns))
    jq_col = (j_col // 8).astype(jnp.bfloat16)
    jr_col = (j_col % 8).astype(jnp.bfloat16)
    jqr = jnp.concatenate([jq_col, jr_col], axis=1)
    PCH = 192
    idx_chunks, val_chunks = [], []
    for p0 in range(0, kc, PCH):
        p_col = jax.lax.broadcasted_iota(jnp.int32, (PCH, 1), 0) + p0
        onehot = rank_row == p_col
        oh_bf = onehot.astype(jnp.bfloat16)
        qr = jax.lax.dot_general(
            oh_bf, jqr, (((1,), (0,)), ((), ())),
            preferred_element_type=jnp.float32)
        idx_chunks.append(
            (jax.lax.slice(qr, (0, 0), (PCH, 1)) * 8.0
             + jax.lax.slice(qr, (0, 1), (PCH, 2))).astype(jnp.int32))
        val_chunks.append(jnp.sum(
            jnp.where(onehot, s_supp_row, 0.0), axis=1, keepdims=True))
    q_iota = jax.lax.broadcasted_iota(jnp.int32, (_NQ, 1), 0) + ns
    s_col = s_row.reshape(n, 1)
    idx_col = jnp.concatenate(idx_chunks + [q_iota], axis=0)
    val_col = jnp.concatenate(
        val_chunks + [jax.lax.slice(s_col, (ns, 0), (n, 1))], axis=0)
    idx_ref[0] = idx_col.reshape(1, m)
    gidx_ref[0] = (idx_col + bb * n).reshape(1, m)

    jn_row = jax.lax.broadcasted_iota(jnp.int32, (1, n), 1)
    RCH = 272
    for r0 in range(0, m, RCH):
        idx_c = jax.lax.slice(idx_col, (r0, 0), (r0 + RCH, 1))
        val_c = jax.lax.slice(val_col, (r0, 0), (r0 + RCH, 1))
        g_c = (idx_c == jn_row).astype(jnp.float32)
        newX_ref[0, pl.ds(r0, RCH), :] = jax.lax.dot_general(
            g_c * val_c, x_ref[0], (((1,), (0,)), ((), ())),
            preferred_element_type=jnp.float32)


def _select(intra, scores, X):
    B, N, D = X.shape
    ns = N - _NQ
    m = ns // 2 + _NQ
    return pl.pallas_call(
        _sel_body,
        grid=(B,),
        in_specs=[
            pl.BlockSpec((1, 1, ns), lambda b_: (b_, 0, 0)),
            pl.BlockSpec((1, 1, N), lambda b_: (b_, 0, 0)),
            pl.BlockSpec((1, N, D), lambda b_: (b_, 0, 0)),
        ],
        out_specs=[
            pl.BlockSpec((1, m, D), lambda b_: (b_, 0, 0)),
            pl.BlockSpec((1, 1, m), lambda b_: (b_, 0, 0)),
            pl.BlockSpec((1, 1, m), lambda b_: (b_, 0, 0)),
        ],
        out_shape=[
            jax.ShapeDtypeStruct((B, m, D), jnp.float32),
            jax.ShapeDtypeStruct((B, 1, m), jnp.int32),
            jax.ShapeDtypeStruct((B, 1, m), jnp.int32),
        ],
        compiler_params=pltpu.CompilerParams(
            dimension_semantics=("arbitrary",)),
    )(intra.reshape(B, 1, ns), scores.reshape(B, 1, N), X)


def _sc_gather(A2, gidx3, idx3):
    BN, N = A2.shape
    B, _, m = idx3.shape
    R = B * m
    info = plsc.get_sparse_core_info()
    NC, NS = info.num_cores, info.num_subcores
    NW = NC * NS
    WPB = NW // B
    NROW = m // WPB
    G = 8
    NCH = NROW // G
    NV = m // _L

    NB = 3
    HMAX = NCH // NB
    NTAIL = NCH - NB * HMAX
    mesh = plsc.VectorSubcoreMesh(core_axis_name="c", subcore_axis_name="s")

    @functools.partial(
        pl.kernel, mesh=mesh,
        out_type=jax.ShapeDtypeStruct((R, m), jnp.float32),
        scratch_types=[
            pltpu.VMEM((m,), jnp.int32),
            pltpu.VMEM((m,), jnp.int32),
            [pltpu.VMEM((G, N), jnp.float32)] * NB,
            [pltpu.VMEM((G, m), jnp.float32)] * NB,
            [pltpu.SemaphoreType.DMA] * NB,
            [pltpu.SemaphoreType.DMA] * NB,
        ],
        compiler_params=pltpu.CompilerParams(needs_layout_passes=False),
    )
    def k(a2, gidx_h, cidx_h, out, colv, rowv, rbs, obs, sgs, sws):
        wid = lax.axis_index("s") * NC + lax.axis_index("c")
        b = wid // WPB
        kk = wid - b * WPB
        pltpu.sync_copy(cidx_h.at[b, 0], colv)
        pltpu.sync_copy(gidx_h.at[b, 0], rowv)

        def g_copy(ci, j):
            return pltpu.make_async_copy(
                a2.at[rowv.at[pl.ds(kk * NROW + ci * G, G)]], rbs[j],
                sgs[j])

        def w_copy(ci, j):
            return pltpu.make_async_copy(
                obs[j], out.at[pl.ds(wid * NROW + ci * G, G)], sws[j])

        def compute(j):
            rb, ob = rbs[j], obs[j]

            def vstep(v, c2):
                cvec = colv[pl.ds(v * _L, _L)]
                vals = [
                    plsc.load_gather(rb, [jnp.full((_L,), i, jnp.int32), cvec])
                    for i in range(G)
                ]
                for i in range(G):
                    ob[i, pl.ds(v * _L, _L)] = vals[i]
                return c2

            lax.fori_loop(0, NV, vstep, 0)

        for j in range(NB):
            g_copy(j, j).start()

        def body(h, carry):
            for j in range(NB):
                a = NB * h + j

                @pl.when(h > 0)
                def _():
                    w_copy(a - NB, j).wait()

                g_copy(a, j).wait()
                compute(j)

                @pl.when(a + NB < NCH)
                def _():
                    g_copy(a + NB, j).start()

                w_copy(a, j).start()
            return carry

        lax.fori_loop(0, HMAX, body, 0)
        for t in range(NTAIL):
            a = NB * HMAX + t
            w_copy(a - NB, t).wait()
            g_copy(a, t).wait()
            compute(t)
            w_copy(a, t).start()
        for t in range(NTAIL):
            w_copy(NB * HMAX + t, t).wait()
        for t in range(NTAIL, NB):
            w_copy(NB * (HMAX - 1) + t, t).wait()

    return k(A2, gidx3, idx3)


def kernel(A, X, W, b):
    B, N, D = X.shape
    ns = N - _NQ
    m = ns // 2 + _NQ
    scores = jax.nn.sigmoid(jnp.squeeze(X @ W + b, -1) / 100.0)
    supp = scores[:, :ns]
    intra = supp - jnp.mean(supp, axis=1, keepdims=True)

    newX, idx3, gidx3 = _select(intra, scores, X)
    newA2 = _sc_gather(A.reshape(B * N, N), gidx3, idx3)
    return newA2.reshape(B, m, m), newX, idx3.reshape(B, m)


# speedup vs baseline: 1.0592x; 1.0592x over previous
import functools

import jax
import jax.numpy as jnp
from jax import lax
from jax.experimental import pallas as pl
from jax.experimental.pallas import tpu as pltpu
from jax.experimental.pallas import tpu_sc as plsc

_NQ = 128
_L = 16


def _sel_body(intra_ref, scores_ref, idx_ref, gidx_ref, val_ref):
    ns = intra_ref.shape[-1]
    n = scores_ref.shape[-1]
    kc = ns // 2
    m = kc + _NQ
    bb = pl.program_id(0)

    it_row = intra_ref[0]
    s_row = scores_ref[0]
    it_col = it_row.reshape(ns, 1)
    j_col = jax.lax.broadcasted_iota(jnp.int32, (ns, 1), 0)
    j_row = jax.lax.broadcasted_iota(jnp.int32, (1, ns), 1)

    CH = 384
    rank_chunks = []
    for c0 in range(0, ns, CH):
        it_i = jax.lax.slice(it_row, (0, c0), (1, c0 + CH))
        i_row = jax.lax.broadcasted_iota(jnp.int32, (1, CH), 1) + c0
        less = it_col < it_i
        tie = (it_col == it_i) & (j_col < i_row)
        mask = (less | tie).astype(jnp.int32)
        rank_chunks.append(jnp.sum(mask, axis=0, keepdims=True))
    rank_row = jnp.concatenate(rank_chunks, axis=1)

    s_supp_row = jax.lax.slice(s_row, (0, 0), (1, ns))
    PCH = 192
    idx_chunks, val_chunks = [], []
    for p0 in range(0, kc, PCH):
        p_col = jax.lax.broadcasted_iota(jnp.int32, (PCH, 1), 0) + p0
        onehot = rank_row == p_col
        idx_chunks.append(jnp.sum(
            jnp.where(onehot, j_row, 0), axis=1, keepdims=True))
        val_chunks.append(jnp.sum(
            jnp.where(onehot, s_supp_row, 0.0), axis=1, keepdims=True))
    q_iota = jax.lax.broadcasted_iota(jnp.int32, (_NQ, 1), 0) + ns
    s_col = s_row.reshape(n, 1)
    idx_col = jnp.concatenate(idx_chunks + [q_iota], axis=0)
    val_col = jnp.concatenate(
        val_chunks + [jax.lax.slice(s_col, (ns, 0), (n, 1))], axis=0)
    idx_ref[0] = idx_col.reshape(1, m)
    gidx_ref[0] = (idx_col + bb * n).reshape(1, m)
    val_ref[0] = val_col.reshape(1, m)


def _select(intra, scores):
    B, ns = intra.shape
    n = scores.shape[-1]
    m = ns // 2 + _NQ
    return pl.pallas_call(
        _sel_body,
        grid=(B,),
        in_specs=[
            pl.BlockSpec((1, 1, ns), lambda b_: (b_, 0, 0)),
            pl.BlockSpec((1, 1, n), lambda b_: (b_, 0, 0)),
        ],
        out_specs=[
            pl.BlockSpec((1, 1, m), lambda b_: (b_, 0, 0)),
            pl.BlockSpec((1, 1, m), lambda b_: (b_, 0, 0)),
            pl.BlockSpec((1, 1, m), lambda b_: (b_, 0, 0)),
        ],
        out_shape=[
            jax.ShapeDtypeStruct((B, 1, m), jnp.int32),
            jax.ShapeDtypeStruct((B, 1, m), jnp.int32),
            jax.ShapeDtypeStruct((B, 1, m), jnp.float32),
        ],
        compiler_params=pltpu.CompilerParams(
            dimension_semantics=("arbitrary",)),
    )(intra.reshape(B, 1, ns), scores.reshape(B, 1, n))


def _newx_body(idx_ref, val_ref, x_ref, newX_ref):
    n = x_ref.shape[-2]
    m = idx_ref.shape[-1]
    idx_col = idx_ref[0].reshape(m, 1)
    val_col = val_ref[0].reshape(m, 1)
    jn_row = jax.lax.broadcasted_iota(jnp.int32, (1, n), 1)
    RCH = 272
    for r0 in range(0, m, RCH):
        idx_c = jax.lax.slice(idx_col, (r0, 0), (r0 + RCH, 1))
        val_c = jax.lax.slice(val_col, (r0, 0), (r0 + RCH, 1))
        g_c = (idx_c == jn_row).astype(jnp.float32)
        newX_ref[0, pl.ds(r0, RCH), :] = jax.lax.dot_general(
            g_c * val_c, x_ref[0], (((1,), (0,)), ((), ())),
            preferred_element_type=jnp.float32)


def _newx(idx3, val3, X):
    B, N, D = X.shape
    m = idx3.shape[-1]
    return pl.pallas_call(
        _newx_body,
        grid=(B,),
        in_specs=[
            pl.BlockSpec((1, 1, m), lambda b_: (b_, 0, 0)),
            pl.BlockSpec((1, 1, m), lambda b_: (b_, 0, 0)),
            pl.BlockSpec((1, N, D), lambda b_: (b_, 0, 0)),
        ],
        out_specs=pl.BlockSpec((1, m, D), lambda b_: (b_, 0, 0)),
        out_shape=jax.ShapeDtypeStruct((B, m, D), jnp.float32),
        compiler_params=pltpu.CompilerParams(
            dimension_semantics=("arbitrary",)),
    )(idx3, val3, X)


def _sc_gather(A2, gidx3, idx3):
    BN, N = A2.shape
    B, _, m = idx3.shape
    R = B * m
    info = plsc.get_sparse_core_info()
    NC, NS = info.num_cores, info.num_subcores
    NW = NC * NS
    WPB = NW // B
    NROW = m // WPB
    G = 8
    NCH = NROW // G
    NV = m // _L

    NB = 3
    HMAX = NCH // NB
    NTAIL = NCH - NB * HMAX
    mesh = plsc.VectorSubcoreMesh(core_axis_name="c", subcore_axis_name="s")

    @functools.partial(
        pl.kernel, mesh=mesh,
        out_type=jax.ShapeDtypeStruct((R, m), jnp.float32),
        scratch_types=[
            pltpu.VMEM((m,), jnp.int32),
            pltpu.VMEM((m,), jnp.int32),
            [pltpu.VMEM((G, N), jnp.float32)] * NB,
            [pltpu.VMEM((G, m), jnp.float32)] * NB,
            [pltpu.SemaphoreType.DMA] * NB,
            [pltpu.SemaphoreType.DMA] * NB,
        ],
        compiler_params=pltpu.CompilerParams(needs_layout_passes=False),
    )
    def k(a2, gidx_h, cidx_h, out, colv, rowv, rbs, obs, sgs, sws):
        wid = lax.axis_index("s") * NC + lax.axis_index("c")
        b = wid // WPB
        kk = wid - b * WPB
        pltpu.sync_copy(cidx_h.at[b, 0], colv)
        pltpu.sync_copy(gidx_h.at[b, 0], rowv)

        def g_copy(ci, j):
            return pltpu.make_async_copy(
                a2.at[rowv.at[pl.ds(kk * NROW + ci * G, G)]], rbs[j],
                sgs[j])

        def w_copy(ci, j):
            return pltpu.make_async_copy(
                obs[j], out.at[pl.ds(wid * NROW + ci * G, G)], sws[j])

        def compute(j):
            rb, ob = rbs[j], obs[j]

            def vstep(v, c2):
                cvec = colv[pl.ds(v * _L, _L)]
                vals = [
                    plsc.load_gather(rb, [jnp.full((_L,), i, jnp.int32), cvec])
                    for i in range(G)
                ]
                for i in range(G):
                    ob[i, pl.ds(v * _L, _L)] = vals[i]
                return c2

            lax.fori_loop(0, NV, vstep, 0)

        for j in range(NB):
            g_copy(j, j).start()

        def body(h, carry):
            for j in range(NB):
                a = NB * h + j

                @pl.when(h > 0)
                def _():
                    w_copy(a - NB, j).wait()

                g_copy(a, j).wait()
                compute(j)

                @pl.when(a + NB < NCH)
                def _():
                    g_copy(a + NB, j).start()

                w_copy(a, j).start()
            return carry

        lax.fori_loop(0, HMAX, body, 0)
        for t in range(NTAIL):
            a = NB * HMAX + t
            w_copy(a - NB, t).wait()
            g_copy(a, t).wait()
            compute(t)
            w_copy(a, t).start()
        for t in range(NTAIL):
            w_copy(NB * HMAX + t, t).wait()
        for t in range(NTAIL, NB):
            w_copy(NB * (HMAX - 1) + t, t).wait()

    return k(A2, gidx3, idx3)


def kernel(A, X, W, b):
    B, N, D = X.shape
    ns = N - _NQ
    m = ns // 2 + _NQ
    scores = jax.nn.sigmoid(jnp.squeeze(X @ W + b, -1) / 100.0)
    supp = scores[:, :ns]
    intra = supp - jnp.mean(supp, axis=1, keepdims=True)

    idx3, gidx3, val3 = _select(intra, scores)
    newA2 = _sc_gather(A.reshape(B * N, N), gidx3, idx3)
    newX = _newx(idx3, val3, X)
    return newA2.reshape(B, m, m), newX, idx3.reshape(B, m)
